# baseline (speedup 1.0000x reference)
import functools

import jax
import jax.numpy as jnp
from jax import lax
from jax.experimental import pallas as pl
from jax.experimental.pallas import tpu as pltpu
from jax.experimental.pallas import tpu_sc as plsc

NUM_RELS = 100000
EMB_DIM = 128
BATCH = 16384


@functools.lru_cache(maxsize=None)
def _make_sc_gather(V, D, B):
    NC, NS = 2, 16
    NW = NC * NS
    assert B % NW == 0
    b_per_w = B // NW
    mesh = plsc.VectorSubcoreMesh(core_axis_name="c", subcore_axis_name="s")

    ch = 128
    nch = b_per_w // ch

    @functools.partial(
        pl.kernel,
        mesh=mesh,
        out_type=jax.ShapeDtypeStruct((B, D), jnp.float32),
        scratch_types=[
            pltpu.VMEM((b_per_w,), jnp.int32),
            pltpu.VMEM((b_per_w, D), jnp.float32),
            pltpu.SemaphoreType.DMA,
            pltpu.SemaphoreType.DMA,
        ],
    )
    def gather_k(table_hbm, idx_hbm, out_hbm, idx_v, rows_v, gsem, wsem):
        wid = lax.axis_index("s") * NC + lax.axis_index("c")
        base = wid * b_per_w
        pltpu.sync_copy(idx_hbm.at[pl.ds(base, b_per_w)], idx_v)
        gathers = []
        for j in range(nch):
            sl = pl.ds(j * ch, ch)
            gathers.append(
                pltpu.async_copy(table_hbm.at[idx_v.at[sl]], rows_v.at[sl], gsem))
        writes = []
        for j in range(nch):
            gathers[j].wait()
            sl = pl.ds(j * ch, ch)
            writes.append(
                pltpu.async_copy(rows_v.at[sl], out_hbm.at[pl.ds(base + j * ch, ch)],
                                 wsem))
        for w in writes:
            w.wait()

    return gather_k


_SIN_COEFFS = (0.9999998622, -0.1666660773, 8.332732438e-3,
               -1.981669233e-4, 2.708326132e-6, -2.069597016e-8)
_COS_COEFFS = (0.9999999739, -0.4999998513, 4.166646236e-2,
               -1.38877318e-3, 2.476905337e-5, -2.70754507e-7,
               1.724375218e-9)


def _poly(y, coeffs):
    acc = coeffs[-1]
    for cf in coeffs[-2::-1]:
        acc = cf + y * acc
    return acc


def _score_body(head_ref, tail_ref, ph_ref, out_ref):
    u = ph_ref[...] - jnp.float32(jnp.pi)
    y = u * u
    su = u * _poly(y, _SIN_COEFFS)
    cu = _poly(y, _COS_COEFFS)
    hr = head_ref[:, :EMB_DIM]
    hi = head_ref[:, EMB_DIM:]
    re = hi * su - hr * cu - tail_ref[:, :EMB_DIM]
    im = hr * su + hi * cu + tail_ref[:, EMB_DIM:]
    dist = jnp.sqrt(re * re + im * im)
    blk = dist.shape[0]
    parts = []
    for j in range(blk // EMB_DIM):
        chunk = dist[j * EMB_DIM:(j + 1) * EMB_DIM, :]
        parts.append(jnp.sum(chunk.T, axis=0))
    out_ref[...] = -jnp.concatenate(parts, axis=0)


def _tc_score(head_emb, tail_emb, phases):
    blk = 8192
    grid = (BATCH // blk,)
    return pl.pallas_call(
        _score_body,
        grid=grid,
        in_specs=[
            pl.BlockSpec((blk, 2 * EMB_DIM), lambda i: (i, 0)),
            pl.BlockSpec((blk, 2 * EMB_DIM), lambda i: (i, 0)),
            pl.BlockSpec((blk, EMB_DIM), lambda i: (i, 0)),
        ],
        out_specs=pl.BlockSpec((blk,), lambda i: (i,)),
        out_shape=jax.ShapeDtypeStruct((BATCH,), jnp.float32),
    )(head_emb, tail_emb, phases)


def kernel(head_emb, tail_emb, rel_table, rel_idx):
    phases = _make_sc_gather(NUM_RELS, EMB_DIM, BATCH)(
        rel_table, rel_idx.astype(jnp.int32))
    return _tc_score(head_emb, tail_emb, phases)


# speedup vs baseline: 1.0358x; 1.0358x over previous
import functools

import jax
import jax.numpy as jnp
from jax import lax
from jax.experimental import pallas as pl
from jax.experimental.pallas import tpu as pltpu
from jax.experimental.pallas import tpu_sc as plsc

NUM_RELS = 100000
EMB_DIM = 128
BATCH = 16384


@functools.lru_cache(maxsize=None)
def _make_sc_gather(V, D, B):
    NC, NS = 2, 16
    NW = NC * NS
    assert B % NW == 0
    b_per_w = B // NW
    mesh = plsc.VectorSubcoreMesh(core_axis_name="c", subcore_axis_name="s")

    ch = 128
    nch = b_per_w // ch

    @functools.partial(
        pl.kernel,
        mesh=mesh,
        out_type=jax.ShapeDtypeStruct((B, D), jnp.float32),
        scratch_types=[
            pltpu.VMEM((b_per_w,), jnp.int32),
            pltpu.VMEM((b_per_w, D), jnp.float32),
            pltpu.SemaphoreType.DMA,
            pltpu.SemaphoreType.DMA,
        ],
    )
    def gather_k(table_hbm, idx_hbm, out_hbm, idx_v, rows_v, gsem, wsem):
        wid = lax.axis_index("s") * NC + lax.axis_index("c")
        base = wid * b_per_w
        pltpu.sync_copy(idx_hbm.at[pl.ds(base, b_per_w)], idx_v)
        gathers = []
        for j in range(nch):
            sl = pl.ds(j * ch, ch)
            gathers.append(
                pltpu.async_copy(table_hbm.at[idx_v.at[sl]], rows_v.at[sl], gsem))
        writes = []
        for j in range(nch):
            gathers[j].wait()
            sl = pl.ds(j * ch, ch)
            writes.append(
                pltpu.async_copy(rows_v.at[sl], out_hbm.at[pl.ds(base + j * ch, ch)],
                                 wsem))
        for w in writes:
            w.wait()

    return gather_k


_SIN_COEFFS = (0.9999998622, -0.1666660773, 8.332732438e-3,
               -1.981669233e-4, 2.708326132e-6, -2.069597016e-8)
_COS_COEFFS = (0.9999999739, -0.4999998513, 4.166646236e-2,
               -1.38877318e-3, 2.476905337e-5, -2.70754507e-7,
               1.724375218e-9)


def _poly(y, coeffs):
    acc = coeffs[-1]
    for cf in coeffs[-2::-1]:
        acc = cf + y * acc
    return acc


def _score_body(head_ref, tail_ref, ph_ref, out_ref):
    u = ph_ref[...] - jnp.float32(jnp.pi)
    y = u * u
    su = u * _poly(y, _SIN_COEFFS)
    cu = _poly(y, _COS_COEFFS)
    hr = head_ref[:, :EMB_DIM]
    hi = head_ref[:, EMB_DIM:]
    re = hi * su - hr * cu - tail_ref[:, :EMB_DIM]
    im = hr * su + hi * cu + tail_ref[:, EMB_DIM:]
    dist = jnp.sqrt(re * re + im * im)
    blk = dist.shape[0]
    parts = []
    for j in range(blk // EMB_DIM):
        chunk = dist[j * EMB_DIM:(j + 1) * EMB_DIM, :]
        parts.append(jnp.sum(chunk.T, axis=0))
    out_ref[...] = -jnp.concatenate(parts, axis=0)


def _tc_score(head_emb, tail_emb, phases):
    blk = 4096
    grid = (BATCH // blk,)
    return pl.pallas_call(
        _score_body,
        grid=grid,
        in_specs=[
            pl.BlockSpec((blk, 2 * EMB_DIM), lambda i: (i, 0)),
            pl.BlockSpec((blk, 2 * EMB_DIM), lambda i: (i, 0)),
            pl.BlockSpec((blk, EMB_DIM), lambda i: (i, 0)),
        ],
        out_specs=pl.BlockSpec((blk,), lambda i: (i,)),
        out_shape=jax.ShapeDtypeStruct((BATCH,), jnp.float32),
    )(head_emb, tail_emb, phases)


def kernel(head_emb, tail_emb, rel_table, rel_idx):
    phases = _make_sc_gather(NUM_RELS, EMB_DIM, BATCH)(
        rel_table, rel_idx.astype(jnp.int32))
    return _tc_score(head_emb, tail_emb, phases)
